# Initial kernel scaffold; baseline (speedup 1.0000x reference)
import jax
import jax.numpy as jnp
from jax.experimental import pallas as pl


def kernel(susceptibility, is_infected, infection_time, transmission_base, edge_agent, edge_venue, beta, noise_u, timer_now, delta_time):
    raise NotImplementedError("write your pallas kernel here")


# speedup vs baseline: 102.5657x; 102.5657x over previous
import functools

import jax
import jax.numpy as jnp
from jax import lax
from jax.experimental import pallas as pl
from jax.experimental.pallas import tpu as pltpu
from jax.experimental.pallas import tpu_sc as plsc

N_A = 100000
N_V = 2000
N_E = 1600000
TAU_C = 0.1

NC = 2
NS = 16
L = 16
NW = NC * NS

A_PAD = 100352
A_ROWS = A_PAD // 128
V_PAD = 2048
E_PER_W = 51200
E_PAD = NW * E_PER_W
CHUNK = 6400
N_CHUNKS = E_PER_W // CHUNK
GROUPS = CHUNK // L
FILL_CHUNK = 2048


def _sc_mesh():
    return plsc.VectorSubcoreMesh(
        core_axis_name="c", subcore_axis_name="s",
        num_cores=NC, num_subcores=NS)


_SC_PARAMS = pltpu.CompilerParams(needs_layout_passes=False)


def _sc_pass1(tb_p, inf_p, ea_p, ev_p):
    @functools.partial(
        pl.kernel,
        out_type=[jax.ShapeDtypeStruct((NW, V_PAD), jnp.float32),
                  jax.ShapeDtypeStruct((NW, V_PAD), jnp.float32)],
        mesh=_sc_mesh(),
        compiler_params=_SC_PARAMS,
        scratch_types=[
            pltpu.VMEM((A_PAD,), jnp.float32),
            pltpu.VMEM((FILL_CHUNK,), jnp.float32),
            pltpu.VMEM((V_PAD,), jnp.float32),
            pltpu.VMEM((V_PAD,), jnp.float32),
            pltpu.VMEM((CHUNK,), jnp.int32),
            pltpu.VMEM((CHUNK,), jnp.int32),
        ],
    )
    def k(tb_hbm, inf_hbm, ea_hbm, ev_hbm, vsum_hbm, vcnt_hbm,
          table, infb, vsum, vcnt, eab, evb):
        wid = lax.axis_index("s") * NC + lax.axis_index("c")
        zero16 = jnp.zeros((L,), jnp.float32)
        one16 = jnp.ones((L,), jnp.float32)

        def fill(i, _):
            off = i * FILL_CHUNK
            pltpu.sync_copy(tb_hbm.at[pl.ds(off, FILL_CHUNK)],
                            table.at[pl.ds(off, FILL_CHUNK)])
            pltpu.sync_copy(inf_hbm.at[pl.ds(off, FILL_CHUNK)], infb)

            def mul(j, _):
                s = j * L
                table[pl.ds(off + s, L)] = (
                    table[pl.ds(off + s, L)] * infb[pl.ds(s, L)])
                return 0
            lax.fori_loop(0, FILL_CHUNK // L, mul, 0)
            return 0
        lax.fori_loop(0, A_PAD // FILL_CHUNK, fill, 0)

        def z(i, _):
            vsum[pl.ds(i * L, L)] = zero16
            vcnt[pl.ds(i * L, L)] = zero16
            return 0
        lax.fori_loop(0, V_PAD // L, z, 0)

        base = wid * E_PER_W
        for c in range(N_CHUNKS):
            off = base + c * CHUNK
            pltpu.sync_copy(ea_hbm.at[pl.ds(off, CHUNK)], eab)
            pltpu.sync_copy(ev_hbm.at[pl.ds(off, CHUNK)], evb)

            def body(t, _):
                s = t * L
                va = eab[pl.ds(s, L)]
                vv = evb[pl.ds(s, L)]
                tv = plsc.load_gather(table, [va])
                plsc.addupdate_scatter(vsum, [vv], tv)
                plsc.addupdate_scatter(vcnt, [vv], one16)
                return 0
            lax.fori_loop(0, GROUPS, body, 0)

        pltpu.sync_copy(vsum, vsum_hbm.at[wid])
        pltpu.sync_copy(vcnt, vcnt_hbm.at[wid])

    return k(tb_p, inf_p, ea_p, ev_p)


def _tc_venue(vsum_p, vcnt_p, beta2):
    def body(vs_ref, vc_ref, b_ref, out_ref):
        s = jnp.sum(vs_ref[...], axis=0, keepdims=True)
        cnt = jnp.sum(vc_ref[...], axis=0, keepdims=True)
        out_ref[...] = b_ref[...] * s / jnp.maximum(cnt, 1.0)

    return pl.pallas_call(
        body,
        out_shape=jax.ShapeDtypeStruct((1, V_PAD), jnp.float32),
    )(vsum_p, vcnt_p, beta2)


def _sc_pass2(vval_p, ea_p, ev_p):
    @functools.partial(
        pl.kernel,
        out_type=jax.ShapeDtypeStruct((NW, A_PAD), jnp.float32),
        mesh=_sc_mesh(),
        compiler_params=_SC_PARAMS,
        scratch_types=[
            pltpu.VMEM((A_PAD,), jnp.float32),
            pltpu.VMEM((V_PAD,), jnp.float32),
            pltpu.VMEM((CHUNK,), jnp.int32),
            pltpu.VMEM((CHUNK,), jnp.int32),
        ],
    )
    def k(vval_hbm, ea_hbm, ev_hbm, expo_hbm, expo, vval, eab, evb):
        wid = lax.axis_index("s") * NC + lax.axis_index("c")
        zero16 = jnp.zeros((L,), jnp.float32)

        pltpu.sync_copy(vval_hbm, vval)

        def z(i, _):
            expo[pl.ds(i * L, L)] = zero16
            return 0
        lax.fori_loop(0, A_PAD // L, z, 0)

        base = wid * E_PER_W
        for c in range(N_CHUNKS):
            off = base + c * CHUNK
            pltpu.sync_copy(ea_hbm.at[pl.ds(off, CHUNK)], eab)
            pltpu.sync_copy(ev_hbm.at[pl.ds(off, CHUNK)], evb)

            def body(t, _):
                s = t * L
                va = eab[pl.ds(s, L)]
                vv = evb[pl.ds(s, L)]
                x = plsc.load_gather(vval, [vv])
                plsc.addupdate_scatter(expo, [va], x)
                return 0
            lax.fori_loop(0, GROUPS, body, 0)

        pltpu.sync_copy(expo, expo_hbm.at[wid])

    return k(vval_p, ea_p, ev_p)


def _tc_final(expo3, s2, inf2, it2, u2, tn2, dt2):
    def body(e_ref, s_ref, i_ref, t_ref, u_ref, tn_ref, dt_ref,
             sus_o, inf_o, it_o, sym_o, nip_o):
        expo = jnp.sum(e_ref[...], axis=0)
        s = s_ref[...]
        infected = i_ref[...]
        itime = t_ref[...]
        u = u_ref[...]
        tn = tn_ref[0, 0]
        dt = dt_ref[0, 0]

        nip = jnp.exp(-dt * s * expo)
        p = jnp.clip(nip, 1e-6, 1.0 - 1e-6)
        a = (jnp.log(p) - jnp.log(-jnp.log(u))) / TAU_C
        b = (jnp.log1p(-p) - jnp.log(-jnp.log(1.0 - u))) / TAU_C
        m = jnp.maximum(a, b)
        ea = jnp.exp(a - m)
        eb = jnp.exp(b - m)
        new_inf = eb / (ea + eb)

        sus_o[...] = jnp.maximum(0.0, s - new_inf)
        inf_o[...] = infected + new_inf
        itn = jnp.where(new_inf > 0.5, tn, itime)
        it_o[...] = itn
        sym_o[...] = (infected + new_inf) * jnp.exp(-(tn - itn))
        nip_o[...] = nip

    shp = jax.ShapeDtypeStruct((A_ROWS, 128), jnp.float32)
    return pl.pallas_call(
        body,
        out_shape=[shp, shp, shp, shp, shp],
    )(expo3, s2, inf2, it2, u2, tn2, dt2)


def kernel(susceptibility, is_infected, infection_time, transmission_base,
           edge_agent, edge_venue, beta, noise_u, timer_now, delta_time):
    f32 = jnp.float32

    def pad_a(x, v):
        return jnp.concatenate(
            [x.astype(f32), jnp.full((A_PAD - N_A,), v, f32)])

    tb_p = pad_a(transmission_base, 0.0)
    inf_p = pad_a(is_infected, 0.0)
    ea_p = jnp.concatenate(
        [edge_agent, jnp.full((E_PAD - N_E,), N_A, jnp.int32)])
    ev_p = jnp.concatenate(
        [edge_venue, jnp.full((E_PAD - N_E,), N_V, jnp.int32)])
    beta2 = jnp.concatenate(
        [beta, jnp.zeros((V_PAD - N_V,), f32)]).reshape(1, V_PAD)

    vsum_p, vcnt_p = _sc_pass1(tb_p, inf_p, ea_p, ev_p)
    vval = _tc_venue(vsum_p, vcnt_p, beta2)
    expo_parts = _sc_pass2(vval.reshape(V_PAD), ea_p, ev_p)

    s2 = pad_a(susceptibility, 0.0).reshape(A_ROWS, 128)
    it2 = pad_a(infection_time, 0.0).reshape(A_ROWS, 128)
    u2 = pad_a(noise_u, 0.5).reshape(A_ROWS, 128)
    inf2 = inf_p.reshape(A_ROWS, 128)
    expo3 = expo_parts.reshape(NW, A_ROWS, 128)
    tn2 = timer_now.astype(f32).reshape(1, 1)
    dt2 = delta_time.astype(f32).reshape(1, 1)

    sus, isi, itn, sym, nip = _tc_final(expo3, s2, inf2, it2, u2, tn2, dt2)
    flat = lambda x: x.reshape(A_PAD)[:N_A]
    return flat(sus), flat(isi), flat(itn), flat(sym), flat(nip)


# speedup vs baseline: 289.5342x; 2.8229x over previous
import functools

import jax
import jax.numpy as jnp
from jax import lax
from jax.experimental import pallas as pl
from jax.experimental.pallas import tpu as pltpu
from jax.experimental.pallas import tpu_sc as plsc

N_A = 100000
N_V = 2000
N_E = 1600000
TAU_C = 0.1

NC = 2
NS = 16
L = 16
NW = NC * NS

A_PAD = 100352
A_ROWS = A_PAD // 128
V_PAD = 2048
E_PER_W = N_E // NW
CHUNK = 2000
N_CHUNKS = E_PER_W // CHUNK
GROUPS = CHUNK // L


def _sc_mesh():
    return plsc.VectorSubcoreMesh(
        core_axis_name="c", subcore_axis_name="s",
        num_cores=NC, num_subcores=NS)


_SC_PARAMS = pltpu.CompilerParams(needs_layout_passes=False)


def _edge_loop(wid, ea_hbm, ev_hbm, eabs, evbs, sems, group_body):
    base = wid * E_PER_W

    def start(c, slot):
        off = base + c * CHUNK
        ha = pltpu.make_async_copy(ea_hbm.at[pl.ds(off, CHUNK)],
                                   eabs[slot], sems[slot])
        hv = pltpu.make_async_copy(ev_hbm.at[pl.ds(off, CHUNK)],
                                   evbs[slot], sems[2 + slot])
        ha.start()
        hv.start()
        return ha, hv

    pending = start(0, 0)
    for c in range(N_CHUNKS):
        slot = c % 2
        pending[0].wait()
        pending[1].wait()
        if c + 1 < N_CHUNKS:
            pending = start(c + 1, 1 - slot)

        def body(t, _):
            s = t * L
            group_body(eabs[slot][pl.ds(s, L)], evbs[slot][pl.ds(s, L)])
            return 0
        lax.fori_loop(0, GROUPS, body, 0, unroll=5)


def _sc_pass1(trans_p, ea, ev):
    @functools.partial(
        pl.kernel,
        out_type=[jax.ShapeDtypeStruct((NW, V_PAD), jnp.float32),
                  jax.ShapeDtypeStruct((NW, V_PAD), jnp.float32)],
        mesh=_sc_mesh(),
        compiler_params=_SC_PARAMS,
        scratch_types=[
            pltpu.VMEM((A_PAD,), jnp.float32),
            pltpu.VMEM((V_PAD,), jnp.float32),
            pltpu.VMEM((V_PAD,), jnp.float32),
            pltpu.VMEM((CHUNK,), jnp.int32),
            pltpu.VMEM((CHUNK,), jnp.int32),
            pltpu.VMEM((CHUNK,), jnp.int32),
            pltpu.VMEM((CHUNK,), jnp.int32),
            pltpu.SemaphoreType.DMA,
            pltpu.SemaphoreType.DMA,
            pltpu.SemaphoreType.DMA,
            pltpu.SemaphoreType.DMA,
        ],
    )
    def k(tr_hbm, ea_hbm, ev_hbm, vsum_hbm, vcnt_hbm,
          table, vsum, vcnt, eab0, eab1, evb0, evb1, s0, s1, s2, s3):
        wid = lax.axis_index("s") * NC + lax.axis_index("c")
        zero16 = jnp.zeros((L,), jnp.float32)
        one16 = jnp.ones((L,), jnp.float32)

        htab = pltpu.make_async_copy(tr_hbm, table, s0)
        htab.start()

        def z(i, _):
            vsum[pl.ds(i * L, L)] = zero16
            vcnt[pl.ds(i * L, L)] = zero16
            return 0
        lax.fori_loop(0, V_PAD // L, z, 0, unroll=8)

        htab.wait()

        def group(va, vv):
            tv = plsc.load_gather(table, [va])
            plsc.addupdate_scatter(vsum, [vv], tv)
            plsc.addupdate_scatter(vcnt, [vv], one16)

        _edge_loop(wid, ea_hbm, ev_hbm, (eab0, eab1), (evb0, evb1),
                   (s0, s1, s2, s3), group)

        pltpu.sync_copy(vsum, vsum_hbm.at[wid])
        pltpu.sync_copy(vcnt, vcnt_hbm.at[wid])

    return k(trans_p, ea, ev)


def _sc_pass2(vval_p, ea, ev):
    @functools.partial(
        pl.kernel,
        out_type=jax.ShapeDtypeStruct((NW, A_PAD), jnp.float32),
        mesh=_sc_mesh(),
        compiler_params=_SC_PARAMS,
        scratch_types=[
            pltpu.VMEM((A_PAD,), jnp.float32),
            pltpu.VMEM((V_PAD,), jnp.float32),
            pltpu.VMEM((CHUNK,), jnp.int32),
            pltpu.VMEM((CHUNK,), jnp.int32),
            pltpu.VMEM((CHUNK,), jnp.int32),
            pltpu.VMEM((CHUNK,), jnp.int32),
            pltpu.SemaphoreType.DMA,
            pltpu.SemaphoreType.DMA,
            pltpu.SemaphoreType.DMA,
            pltpu.SemaphoreType.DMA,
        ],
    )
    def k(vval_hbm, ea_hbm, ev_hbm, expo_hbm,
          expo, vval, eab0, eab1, evb0, evb1, s0, s1, s2, s3):
        wid = lax.axis_index("s") * NC + lax.axis_index("c")
        zero16 = jnp.zeros((L,), jnp.float32)

        hv = pltpu.make_async_copy(vval_hbm, vval, s0)
        hv.start()

        def z(i, _):
            expo[pl.ds(i * L, L)] = zero16
            return 0
        lax.fori_loop(0, A_PAD // L, z, 0, unroll=8)

        hv.wait()

        def group(va, vv):
            x = plsc.load_gather(vval, [vv])
            plsc.addupdate_scatter(expo, [va], x)

        _edge_loop(wid, ea_hbm, ev_hbm, (eab0, eab1), (evb0, evb1),
                   (s0, s1, s2, s3), group)

        pltpu.sync_copy(expo, expo_hbm.at[wid])

    return k(vval_p, ea, ev)


def _tc_trans(tb2, inf2):
    def body(tb_ref, inf_ref, out_ref):
        out_ref[...] = tb_ref[...] * inf_ref[...]

    return pl.pallas_call(
        body,
        out_shape=jax.ShapeDtypeStruct((A_ROWS, 128), jnp.float32),
    )(tb2, inf2)


def _tc_venue(vsum_p, vcnt_p, beta2):
    def body(vs_ref, vc_ref, b_ref, out_ref):
        s = jnp.sum(vs_ref[...], axis=0, keepdims=True)
        cnt = jnp.sum(vc_ref[...], axis=0, keepdims=True)
        out_ref[...] = b_ref[...] * s / jnp.maximum(cnt, 1.0)

    return pl.pallas_call(
        body,
        out_shape=jax.ShapeDtypeStruct((1, V_PAD), jnp.float32),
    )(vsum_p, vcnt_p, beta2)


def _tc_final(expo3, s2, inf2, it2, u2, tn2, dt2):
    def body(e_ref, s_ref, i_ref, t_ref, u_ref, tn_ref, dt_ref,
             sus_o, inf_o, it_o, sym_o, nip_o):
        expo = jnp.sum(e_ref[...], axis=0)
        s = s_ref[...]
        infected = i_ref[...]
        itime = t_ref[...]
        u = u_ref[...]
        tn = tn_ref[0, 0]
        dt = dt_ref[0, 0]

        nip = jnp.exp(-dt * s * expo)
        p = jnp.clip(nip, 1e-6, 1.0 - 1e-6)
        a = (jnp.log(p) - jnp.log(-jnp.log(u))) / TAU_C
        b = (jnp.log1p(-p) - jnp.log(-jnp.log(1.0 - u))) / TAU_C
        m = jnp.maximum(a, b)
        ea = jnp.exp(a - m)
        eb = jnp.exp(b - m)
        new_inf = eb / (ea + eb)

        sus_o[...] = jnp.maximum(0.0, s - new_inf)
        inf_o[...] = infected + new_inf
        itn = jnp.where(new_inf > 0.5, tn, itime)
        it_o[...] = itn
        sym_o[...] = (infected + new_inf) * jnp.exp(-(tn - itn))
        nip_o[...] = nip

    shp = jax.ShapeDtypeStruct((A_ROWS, 128), jnp.float32)
    return pl.pallas_call(
        body,
        out_shape=[shp, shp, shp, shp, shp],
    )(expo3, s2, inf2, it2, u2, tn2, dt2)


def kernel(susceptibility, is_infected, infection_time, transmission_base,
           edge_agent, edge_venue, beta, noise_u, timer_now, delta_time):
    f32 = jnp.float32

    def pad_a(x, v):
        return jnp.concatenate(
            [x.astype(f32), jnp.full((A_PAD - N_A,), v, f32)]
        ).reshape(A_ROWS, 128)

    tb2 = pad_a(transmission_base, 0.0)
    inf2 = pad_a(is_infected, 0.0)
    beta2 = jnp.concatenate(
        [beta, jnp.zeros((V_PAD - N_V,), f32)]).reshape(1, V_PAD)

    trans_p = _tc_trans(tb2, inf2).reshape(A_PAD)
    vsum_p, vcnt_p = _sc_pass1(trans_p, edge_agent, edge_venue)
    vval = _tc_venue(vsum_p, vcnt_p, beta2)
    expo_parts = _sc_pass2(vval.reshape(V_PAD), edge_agent, edge_venue)

    s2 = pad_a(susceptibility, 0.0)
    it2 = pad_a(infection_time, 0.0)
    u2 = pad_a(noise_u, 0.5)
    expo3 = expo_parts.reshape(NW, A_ROWS, 128)
    tn2 = timer_now.astype(f32).reshape(1, 1)
    dt2 = delta_time.astype(f32).reshape(1, 1)

    sus, isi, itn, sym, nip = _tc_final(expo3, s2, inf2, it2, u2, tn2, dt2)
    flat = lambda x: x.reshape(A_PAD)[:N_A]
    return flat(sus), flat(isi), flat(itn), flat(sym), flat(nip)


# speedup vs baseline: 324.7523x; 1.1216x over previous
import functools

import jax
import jax.numpy as jnp
from jax import lax
from jax.experimental import pallas as pl
from jax.experimental.pallas import tpu as pltpu
from jax.experimental.pallas import tpu_sc as plsc

N_A = 100000
N_V = 2000
N_E = 1600000
TAU_C = 0.1

NC = 2
NS = 16
L = 16
NW = NC * NS

A_PAD = 100352
A_ROWS = A_PAD // 128
V_PAD = 2048
E_PER_W = N_E // NW
CHUNK = 2000
N_CHUNKS = E_PER_W // CHUNK
GROUPS = CHUNK // L


def _sc_mesh():
    return plsc.VectorSubcoreMesh(
        core_axis_name="c", subcore_axis_name="s",
        num_cores=NC, num_subcores=NS)


_SC_PARAMS = pltpu.CompilerParams(needs_layout_passes=False)


def _edge_loop(wid, ea_hbm, ev_hbm, eabs, evbs, sems, group_body):
    base = wid * E_PER_W

    def start(c, slot):
        off = base + c * CHUNK
        ha = pltpu.make_async_copy(ea_hbm.at[pl.ds(off, CHUNK)],
                                   eabs[slot], sems[slot])
        hv = pltpu.make_async_copy(ev_hbm.at[pl.ds(off, CHUNK)],
                                   evbs[slot], sems[2 + slot])
        ha.start()
        hv.start()
        return ha, hv

    pending = start(0, 0)
    for c in range(N_CHUNKS):
        slot = c % 2
        pending[0].wait()
        pending[1].wait()
        if c + 1 < N_CHUNKS:
            pending = start(c + 1, 1 - slot)

        def body(t, _):
            s = t * L
            group_body(eabs[slot][pl.ds(s, L)], evbs[slot][pl.ds(s, L)])
            return 0
        lax.fori_loop(0, GROUPS, body, 0, unroll=5)


def _sc_pass1(trans_p, ea, ev):
    @functools.partial(
        pl.kernel,
        out_type=[jax.ShapeDtypeStruct((NW, V_PAD), jnp.float32),
                  jax.ShapeDtypeStruct((NW, V_PAD), jnp.float32)],
        mesh=_sc_mesh(),
        compiler_params=_SC_PARAMS,
        scratch_types=[
            pltpu.VMEM((A_PAD,), jnp.float32),
            pltpu.VMEM((V_PAD,), jnp.float32),
            pltpu.VMEM((V_PAD,), jnp.float32),
            pltpu.VMEM((CHUNK,), jnp.int32),
            pltpu.VMEM((CHUNK,), jnp.int32),
            pltpu.VMEM((CHUNK,), jnp.int32),
            pltpu.VMEM((CHUNK,), jnp.int32),
            pltpu.SemaphoreType.DMA,
            pltpu.SemaphoreType.DMA,
            pltpu.SemaphoreType.DMA,
            pltpu.SemaphoreType.DMA,
        ],
    )
    def k(tr_hbm, ea_hbm, ev_hbm, vsum_hbm, vcnt_hbm,
          table, vsum, vcnt, eab0, eab1, evb0, evb1, s0, s1, s2, s3):
        wid = lax.axis_index("s") * NC + lax.axis_index("c")
        zero16 = jnp.zeros((L,), jnp.float32)
        one16 = jnp.ones((L,), jnp.float32)

        htab = pltpu.make_async_copy(tr_hbm, table, s0)
        htab.start()

        def z(i, _):
            vsum[pl.ds(i * L, L)] = zero16
            vcnt[pl.ds(i * L, L)] = zero16
            return 0
        lax.fori_loop(0, V_PAD // L, z, 0, unroll=8)

        htab.wait()

        def group(va, vv):
            tv = plsc.load_gather(table, [va])
            plsc.addupdate_scatter(vsum, [vv], tv)
            plsc.addupdate_scatter(vcnt, [vv], one16)

        _edge_loop(wid, ea_hbm, ev_hbm, (eab0, eab1), (evb0, evb1),
                   (s0, s1, s2, s3), group)

        pltpu.sync_copy(vsum, vsum_hbm.at[wid])
        pltpu.sync_copy(vcnt, vcnt_hbm.at[wid])

    return k(trans_p, ea, ev)


def _sc_pass2(vval_p, ea, ev):
    @functools.partial(
        pl.kernel,
        out_type=jax.ShapeDtypeStruct((NW, A_ROWS, 128), jnp.float32),
        mesh=_sc_mesh(),
        compiler_params=_SC_PARAMS,
        scratch_types=[
            pltpu.VMEM((A_ROWS, 128), jnp.float32),
            pltpu.VMEM((V_PAD,), jnp.float32),
            pltpu.VMEM((CHUNK,), jnp.int32),
            pltpu.VMEM((CHUNK,), jnp.int32),
            pltpu.VMEM((CHUNK,), jnp.int32),
            pltpu.VMEM((CHUNK,), jnp.int32),
            pltpu.SemaphoreType.DMA,
            pltpu.SemaphoreType.DMA,
            pltpu.SemaphoreType.DMA,
            pltpu.SemaphoreType.DMA,
        ],
    )
    def k(vval_hbm, ea_hbm, ev_hbm, expo_hbm,
          expo, vval, eab0, eab1, evb0, evb1, s0, s1, s2, s3):
        wid = lax.axis_index("s") * NC + lax.axis_index("c")
        zero16 = jnp.zeros((L,), jnp.float32)

        hv = pltpu.make_async_copy(vval_hbm, vval, s0)
        hv.start()

        def zrow(r, _):
            def zc(j, _):
                expo[r, pl.ds(j * L, L)] = zero16
                return 0
            lax.fori_loop(0, 128 // L, zc, 0, unroll=8)
            return 0
        lax.fori_loop(0, A_ROWS, zrow, 0)

        hv.wait()

        def group(va, vv):
            x = plsc.load_gather(vval, [vv])
            row = lax.shift_right_logical(va, 7)
            col = lax.bitwise_and(va, 127)
            plsc.addupdate_scatter(expo, [row, col], x)

        _edge_loop(wid, ea_hbm, ev_hbm, (eab0, eab1), (evb0, evb1),
                   (s0, s1, s2, s3), group)

        pltpu.sync_copy(expo, expo_hbm.at[wid])

    return k(vval_p, ea, ev)


def _tc_trans(tb2, inf2):
    def body(tb_ref, inf_ref, out_ref):
        out_ref[...] = tb_ref[...] * inf_ref[...]

    return pl.pallas_call(
        body,
        out_shape=jax.ShapeDtypeStruct((A_ROWS, 128), jnp.float32),
    )(tb2, inf2)


def _tc_venue(vsum_p, vcnt_p, beta2):
    def body(vs_ref, vc_ref, b_ref, out_ref):
        s = jnp.sum(vs_ref[...], axis=0, keepdims=True)
        cnt = jnp.sum(vc_ref[...], axis=0, keepdims=True)
        out_ref[...] = b_ref[...] * s / jnp.maximum(cnt, 1.0)

    return pl.pallas_call(
        body,
        out_shape=jax.ShapeDtypeStruct((1, V_PAD), jnp.float32),
    )(vsum_p, vcnt_p, beta2)


def _tc_final(expo3, s2, inf2, it2, u2, tn2, dt2):
    def body(e_ref, s_ref, i_ref, t_ref, u_ref, tn_ref, dt_ref,
             sus_o, inf_o, it_o, sym_o, nip_o):
        expo = jnp.sum(e_ref[...], axis=0)
        s = s_ref[...]
        infected = i_ref[...]
        itime = t_ref[...]
        u = u_ref[...]
        tn = tn_ref[0, 0]
        dt = dt_ref[0, 0]

        nip = jnp.exp(-dt * s * expo)
        p = jnp.clip(nip, 1e-6, 1.0 - 1e-6)
        a = (jnp.log(p) - jnp.log(-jnp.log(u))) / TAU_C
        b = (jnp.log1p(-p) - jnp.log(-jnp.log(1.0 - u))) / TAU_C
        m = jnp.maximum(a, b)
        ea = jnp.exp(a - m)
        eb = jnp.exp(b - m)
        new_inf = eb / (ea + eb)

        sus_o[...] = jnp.maximum(0.0, s - new_inf)
        inf_o[...] = infected + new_inf
        itn = jnp.where(new_inf > 0.5, tn, itime)
        it_o[...] = itn
        sym_o[...] = (infected + new_inf) * jnp.exp(-(tn - itn))
        nip_o[...] = nip

    shp = jax.ShapeDtypeStruct((A_ROWS, 128), jnp.float32)
    return pl.pallas_call(
        body,
        out_shape=[shp, shp, shp, shp, shp],
    )(expo3, s2, inf2, it2, u2, tn2, dt2)


def kernel(susceptibility, is_infected, infection_time, transmission_base,
           edge_agent, edge_venue, beta, noise_u, timer_now, delta_time):
    f32 = jnp.float32

    def pad_a(x, v):
        return jnp.concatenate(
            [x.astype(f32), jnp.full((A_PAD - N_A,), v, f32)]
        ).reshape(A_ROWS, 128)

    tb2 = pad_a(transmission_base, 0.0)
    inf2 = pad_a(is_infected, 0.0)
    beta2 = jnp.concatenate(
        [beta, jnp.zeros((V_PAD - N_V,), f32)]).reshape(1, V_PAD)

    trans_p = _tc_trans(tb2, inf2).reshape(A_PAD)
    vsum_p, vcnt_p = _sc_pass1(trans_p, edge_agent, edge_venue)
    vval = _tc_venue(vsum_p, vcnt_p, beta2)
    expo_parts = _sc_pass2(vval.reshape(V_PAD), edge_agent, edge_venue)

    s2 = pad_a(susceptibility, 0.0)
    it2 = pad_a(infection_time, 0.0)
    u2 = pad_a(noise_u, 0.5)
    expo3 = expo_parts
    tn2 = timer_now.astype(f32).reshape(1, 1)
    dt2 = delta_time.astype(f32).reshape(1, 1)

    sus, isi, itn, sym, nip = _tc_final(expo3, s2, inf2, it2, u2, tn2, dt2)
    flat = lambda x: x.reshape(A_PAD)[:N_A]
    return flat(sus), flat(isi), flat(itn), flat(sym), flat(nip)


# speedup vs baseline: 379.2022x; 1.1677x over previous
import functools

import jax
import jax.numpy as jnp
from jax import lax
from jax.experimental import pallas as pl
from jax.experimental.pallas import tpu as pltpu
from jax.experimental.pallas import tpu_sc as plsc

N_A = 100000
N_V = 2000
N_E = 1600000
TAU_C = 0.1

NC = 2
NS = 16
L = 16
NW = NC * NS

A_PAD = 100352
A_ROWS = A_PAD // 128
V_PAD = 2048
E_PER_W = N_E // NW
CHUNK = 2000
N_CHUNKS = E_PER_W // CHUNK
GROUPS = CHUNK // L
GBATCH = 5


def _sc_mesh():
    return plsc.VectorSubcoreMesh(
        core_axis_name="c", subcore_axis_name="s",
        num_cores=NC, num_subcores=NS)


_SC_PARAMS = pltpu.CompilerParams(needs_layout_passes=False)


def _edge_loop(wid, ea_hbm, ev_hbm, eabs, evbs, sems, group_body):
    base = wid * E_PER_W

    def start(c, slot):
        off = base + c * CHUNK
        ha = pltpu.make_async_copy(ea_hbm.at[pl.ds(off, CHUNK)],
                                   eabs[slot], sems[slot])
        hv = pltpu.make_async_copy(ev_hbm.at[pl.ds(off, CHUNK)],
                                   evbs[slot], sems[2 + slot])
        ha.start()
        hv.start()
        return ha, hv

    pending = start(0, 0)
    for c in range(N_CHUNKS):
        slot = c % 2
        pending[0].wait()
        pending[1].wait()
        if c + 1 < N_CHUNKS:
            pending = start(c + 1, 1 - slot)

        def body(t, _):
            s = t * (L * GBATCH)
            vas = [eabs[slot][pl.ds(s + g * L, L)] for g in range(GBATCH)]
            vvs = [evbs[slot][pl.ds(s + g * L, L)] for g in range(GBATCH)]
            group_body(vas, vvs)
            return 0
        lax.fori_loop(0, GROUPS // GBATCH, body, 0, unroll=2)


def _sc_pass1(trans_p, ea, ev):
    @functools.partial(
        pl.kernel,
        out_type=[jax.ShapeDtypeStruct((NW, V_PAD), jnp.float32),
                  jax.ShapeDtypeStruct((NW, V_PAD), jnp.float32)],
        mesh=_sc_mesh(),
        compiler_params=_SC_PARAMS,
        scratch_types=[
            pltpu.VMEM((A_PAD,), jnp.float32),
            pltpu.VMEM((V_PAD,), jnp.float32),
            pltpu.VMEM((V_PAD,), jnp.float32),
            pltpu.VMEM((CHUNK,), jnp.int32),
            pltpu.VMEM((CHUNK,), jnp.int32),
            pltpu.VMEM((CHUNK,), jnp.int32),
            pltpu.VMEM((CHUNK,), jnp.int32),
            pltpu.SemaphoreType.DMA,
            pltpu.SemaphoreType.DMA,
            pltpu.SemaphoreType.DMA,
            pltpu.SemaphoreType.DMA,
        ],
    )
    def k(tr_hbm, ea_hbm, ev_hbm, vsum_hbm, vcnt_hbm,
          table, vsum, vcnt, eab0, eab1, evb0, evb1, s0, s1, s2, s3):
        wid = lax.axis_index("s") * NC + lax.axis_index("c")
        zero16 = jnp.zeros((L,), jnp.float32)
        one16 = jnp.ones((L,), jnp.float32)

        htab = pltpu.make_async_copy(tr_hbm, table, s0)
        htab.start()

        def z(i, _):
            vsum[pl.ds(i * L, L)] = zero16
            vcnt[pl.ds(i * L, L)] = zero16
            return 0
        lax.fori_loop(0, V_PAD // L, z, 0, unroll=8)

        htab.wait()

        def group(vas, vvs):
            tvs = [plsc.load_gather(table, [va]) for va in vas]
            for vv, tv in zip(vvs, tvs):
                plsc.addupdate_scatter(vsum, [vv], tv)
            for vv in vvs:
                plsc.addupdate_scatter(vcnt, [vv], one16)

        _edge_loop(wid, ea_hbm, ev_hbm, (eab0, eab1), (evb0, evb1),
                   (s0, s1, s2, s3), group)

        pltpu.sync_copy(vsum, vsum_hbm.at[wid])
        pltpu.sync_copy(vcnt, vcnt_hbm.at[wid])

    return k(trans_p, ea, ev)


def _sc_pass2(vval_p, ea, ev):
    @functools.partial(
        pl.kernel,
        out_type=jax.ShapeDtypeStruct((NW, A_ROWS, 128), jnp.float32),
        mesh=_sc_mesh(),
        compiler_params=_SC_PARAMS,
        scratch_types=[
            pltpu.VMEM((A_ROWS, 128), jnp.float32),
            pltpu.VMEM((V_PAD,), jnp.float32),
            pltpu.VMEM((CHUNK,), jnp.int32),
            pltpu.VMEM((CHUNK,), jnp.int32),
            pltpu.VMEM((CHUNK,), jnp.int32),
            pltpu.VMEM((CHUNK,), jnp.int32),
            pltpu.SemaphoreType.DMA,
            pltpu.SemaphoreType.DMA,
            pltpu.SemaphoreType.DMA,
            pltpu.SemaphoreType.DMA,
        ],
    )
    def k(vval_hbm, ea_hbm, ev_hbm, expo_hbm,
          expo, vval, eab0, eab1, evb0, evb1, s0, s1, s2, s3):
        wid = lax.axis_index("s") * NC + lax.axis_index("c")
        zero16 = jnp.zeros((L,), jnp.float32)

        hv = pltpu.make_async_copy(vval_hbm, vval, s0)
        hv.start()

        def zrow(r, _):
            def zc(j, _):
                expo[r, pl.ds(j * L, L)] = zero16
                return 0
            lax.fori_loop(0, 128 // L, zc, 0, unroll=8)
            return 0
        lax.fori_loop(0, A_ROWS, zrow, 0)

        hv.wait()

        def group(vas, vvs):
            xs = [plsc.load_gather(vval, [vv]) for vv in vvs]
            rows = [lax.shift_right_logical(va, 7) for va in vas]
            cols = [lax.bitwise_and(va, 127) for va in vas]
            for row, col, x in zip(rows, cols, xs):
                plsc.addupdate_scatter(expo, [row, col], x)

        _edge_loop(wid, ea_hbm, ev_hbm, (eab0, eab1), (evb0, evb1),
                   (s0, s1, s2, s3), group)

        pltpu.sync_copy(expo, expo_hbm.at[wid])

    return k(vval_p, ea, ev)


def _tc_trans(tb2, inf2):
    def body(tb_ref, inf_ref, out_ref):
        out_ref[...] = tb_ref[...] * inf_ref[...]

    return pl.pallas_call(
        body,
        out_shape=jax.ShapeDtypeStruct((A_ROWS, 128), jnp.float32),
    )(tb2, inf2)


def _tc_venue(vsum_p, vcnt_p, beta2):
    def body(vs_ref, vc_ref, b_ref, out_ref):
        s = jnp.sum(vs_ref[...], axis=0, keepdims=True)
        cnt = jnp.sum(vc_ref[...], axis=0, keepdims=True)
        out_ref[...] = b_ref[...] * s / jnp.maximum(cnt, 1.0)

    return pl.pallas_call(
        body,
        out_shape=jax.ShapeDtypeStruct((1, V_PAD), jnp.float32),
    )(vsum_p, vcnt_p, beta2)


def _tc_final(expo3, s2, inf2, it2, u2, tn2, dt2):
    def body(e_ref, s_ref, i_ref, t_ref, u_ref, tn_ref, dt_ref,
             sus_o, inf_o, it_o, sym_o, nip_o):
        expo = jnp.sum(e_ref[...], axis=0)
        s = s_ref[...]
        infected = i_ref[...]
        itime = t_ref[...]
        u = u_ref[...]
        tn = tn_ref[0, 0]
        dt = dt_ref[0, 0]

        nip = jnp.exp(-dt * s * expo)
        p = jnp.clip(nip, 1e-6, 1.0 - 1e-6)
        a = (jnp.log(p) - jnp.log(-jnp.log(u))) / TAU_C
        b = (jnp.log1p(-p) - jnp.log(-jnp.log(1.0 - u))) / TAU_C
        m = jnp.maximum(a, b)
        ea = jnp.exp(a - m)
        eb = jnp.exp(b - m)
        new_inf = eb / (ea + eb)

        sus_o[...] = jnp.maximum(0.0, s - new_inf)
        inf_o[...] = infected + new_inf
        itn = jnp.where(new_inf > 0.5, tn, itime)
        it_o[...] = itn
        sym_o[...] = (infected + new_inf) * jnp.exp(-(tn - itn))
        nip_o[...] = nip

    shp = jax.ShapeDtypeStruct((A_ROWS, 128), jnp.float32)
    return pl.pallas_call(
        body,
        out_shape=[shp, shp, shp, shp, shp],
    )(expo3, s2, inf2, it2, u2, tn2, dt2)


def kernel(susceptibility, is_infected, infection_time, transmission_base,
           edge_agent, edge_venue, beta, noise_u, timer_now, delta_time):
    f32 = jnp.float32

    def pad_a(x, v):
        return jnp.concatenate(
            [x.astype(f32), jnp.full((A_PAD - N_A,), v, f32)]
        ).reshape(A_ROWS, 128)

    tb2 = pad_a(transmission_base, 0.0)
    inf2 = pad_a(is_infected, 0.0)
    beta2 = jnp.concatenate(
        [beta, jnp.zeros((V_PAD - N_V,), f32)]).reshape(1, V_PAD)

    trans_p = _tc_trans(tb2, inf2).reshape(A_PAD)
    vsum_p, vcnt_p = _sc_pass1(trans_p, edge_agent, edge_venue)
    vval = _tc_venue(vsum_p, vcnt_p, beta2)
    expo_parts = _sc_pass2(vval.reshape(V_PAD), edge_agent, edge_venue)

    s2 = pad_a(susceptibility, 0.0)
    it2 = pad_a(infection_time, 0.0)
    u2 = pad_a(noise_u, 0.5)
    expo3 = expo_parts
    tn2 = timer_now.astype(f32).reshape(1, 1)
    dt2 = delta_time.astype(f32).reshape(1, 1)

    sus, isi, itn, sym, nip = _tc_final(expo3, s2, inf2, it2, u2, tn2, dt2)
    flat = lambda x: x.reshape(A_PAD)[:N_A]
    return flat(sus), flat(isi), flat(itn), flat(sym), flat(nip)


# speedup vs baseline: 385.6715x; 1.0171x over previous
import functools

import jax
import jax.numpy as jnp
from jax import lax
from jax.experimental import pallas as pl
from jax.experimental.pallas import tpu as pltpu
from jax.experimental.pallas import tpu_sc as plsc

N_A = 100000
N_V = 2000
N_E = 1600000
TAU_C = 0.1

NC = 2
NS = 16
L = 16
NW = NC * NS

A_PAD = 100352
A_ROWS = A_PAD // 128
V_PAD = 2048
E_PER_W = N_E // NW
CHUNK = 2000
N_CHUNKS = E_PER_W // CHUNK
GROUPS = CHUNK // L
GBATCH = 5


def _sc_mesh():
    return plsc.VectorSubcoreMesh(
        core_axis_name="c", subcore_axis_name="s",
        num_cores=NC, num_subcores=NS)


_SC_PARAMS = pltpu.CompilerParams(needs_layout_passes=False)


def _edge_start(wid, ea_hbm, ev_hbm, eabs, evbs, sems, c, slot):
    off = wid * E_PER_W + c * CHUNK
    ha = pltpu.make_async_copy(ea_hbm.at[pl.ds(off, CHUNK)],
                               eabs[slot], sems[slot])
    hv = pltpu.make_async_copy(ev_hbm.at[pl.ds(off, CHUNK)],
                               evbs[slot], sems[2 + slot])
    ha.start()
    hv.start()
    return ha, hv


def _edge_loop(wid, ea_hbm, ev_hbm, eabs, evbs, sems, group_body, pending):
    def start(c, slot):
        return _edge_start(wid, ea_hbm, ev_hbm, eabs, evbs, sems, c, slot)

    for c in range(N_CHUNKS):
        slot = c % 2
        pending[0].wait()
        pending[1].wait()
        if c + 1 < N_CHUNKS:
            pending = start(c + 1, 1 - slot)

        @plsc.parallel_loop(0, GROUPS // GBATCH, unroll=2)
        def _loop(t):
            s = t * (L * GBATCH)
            vas = [eabs[slot][pl.ds(s + g * L, L)] for g in range(GBATCH)]
            vvs = [evbs[slot][pl.ds(s + g * L, L)] for g in range(GBATCH)]
            group_body(vas, vvs)


def _sc_pass1(trans_p, ea, ev):
    @functools.partial(
        pl.kernel,
        out_type=[jax.ShapeDtypeStruct((NW, V_PAD), jnp.float32),
                  jax.ShapeDtypeStruct((NW, V_PAD), jnp.float32)],
        mesh=_sc_mesh(),
        compiler_params=_SC_PARAMS,
        scratch_types=[
            pltpu.VMEM((A_PAD,), jnp.float32),
            pltpu.VMEM((V_PAD,), jnp.float32),
            pltpu.VMEM((V_PAD,), jnp.float32),
            pltpu.VMEM((CHUNK,), jnp.int32),
            pltpu.VMEM((CHUNK,), jnp.int32),
            pltpu.VMEM((CHUNK,), jnp.int32),
            pltpu.VMEM((CHUNK,), jnp.int32),
            pltpu.SemaphoreType.DMA,
            pltpu.SemaphoreType.DMA,
            pltpu.SemaphoreType.DMA,
            pltpu.SemaphoreType.DMA,
        ],
    )
    def k(tr_hbm, ea_hbm, ev_hbm, vsum_hbm, vcnt_hbm,
          table, vsum, vcnt, eab0, eab1, evb0, evb1, s0, s1, s2, s3):
        wid = lax.axis_index("s") * NC + lax.axis_index("c")
        zero16 = jnp.zeros((L,), jnp.float32)
        one16 = jnp.ones((L,), jnp.float32)

        htab = pltpu.make_async_copy(tr_hbm, table, s1)
        htab.start()
        sems = (s0, s1, s2, s3)
        ebufs = ((eab0, eab1), (evb0, evb1))
        pending = _edge_start(wid, ea_hbm, ev_hbm, ebufs[0], ebufs[1],
                              sems, 0, 0)

        def z(i, _):
            vsum[pl.ds(i * L, L)] = zero16
            vcnt[pl.ds(i * L, L)] = zero16
            return 0
        lax.fori_loop(0, V_PAD // L, z, 0, unroll=8)

        htab.wait()

        def group(vas, vvs):
            tvs = [plsc.load_gather(table, [va]) for va in vas]
            for vv, tv in zip(vvs, tvs):
                plsc.addupdate_scatter(vsum, [vv], tv)
            for vv in vvs:
                plsc.addupdate_scatter(vcnt, [vv], one16)

        _edge_loop(wid, ea_hbm, ev_hbm, ebufs[0], ebufs[1],
                   sems, group, pending)

        pltpu.sync_copy(vsum, vsum_hbm.at[wid])
        pltpu.sync_copy(vcnt, vcnt_hbm.at[wid])

    return k(trans_p, ea, ev)


def _sc_pass2(vval_p, ea, ev):
    @functools.partial(
        pl.kernel,
        out_type=jax.ShapeDtypeStruct((NW, A_ROWS, 128), jnp.float32),
        mesh=_sc_mesh(),
        compiler_params=_SC_PARAMS,
        scratch_types=[
            pltpu.VMEM((A_ROWS, 128), jnp.float32),
            pltpu.VMEM((V_PAD,), jnp.float32),
            pltpu.VMEM((CHUNK,), jnp.int32),
            pltpu.VMEM((CHUNK,), jnp.int32),
            pltpu.VMEM((CHUNK,), jnp.int32),
            pltpu.VMEM((CHUNK,), jnp.int32),
            pltpu.SemaphoreType.DMA,
            pltpu.SemaphoreType.DMA,
            pltpu.SemaphoreType.DMA,
            pltpu.SemaphoreType.DMA,
        ],
    )
    def k(vval_hbm, ea_hbm, ev_hbm, expo_hbm,
          expo, vval, eab0, eab1, evb0, evb1, s0, s1, s2, s3):
        wid = lax.axis_index("s") * NC + lax.axis_index("c")
        zero16 = jnp.zeros((L,), jnp.float32)

        hv = pltpu.make_async_copy(vval_hbm, vval, s1)
        hv.start()
        sems = (s0, s1, s2, s3)
        ebufs = ((eab0, eab1), (evb0, evb1))
        pending = _edge_start(wid, ea_hbm, ev_hbm, ebufs[0], ebufs[1],
                              sems, 0, 0)

        def zrow(r, _):
            def zc(j, _):
                expo[r, pl.ds(j * L, L)] = zero16
                return 0
            lax.fori_loop(0, 128 // L, zc, 0, unroll=8)
            return 0
        lax.fori_loop(0, A_ROWS, zrow, 0, unroll=2)

        hv.wait()

        def group(vas, vvs):
            xs = [plsc.load_gather(vval, [vv]) for vv in vvs]
            rows = [lax.shift_right_logical(va, 7) for va in vas]
            cols = [lax.bitwise_and(va, 127) for va in vas]
            for row, col, x in zip(rows, cols, xs):
                plsc.addupdate_scatter(expo, [row, col], x)

        _edge_loop(wid, ea_hbm, ev_hbm, ebufs[0], ebufs[1],
                   sems, group, pending)

        pltpu.sync_copy(expo, expo_hbm.at[wid])

    return k(vval_p, ea, ev)


def _tc_trans(tb2, inf2):
    def body(tb_ref, inf_ref, out_ref):
        out_ref[...] = tb_ref[...] * inf_ref[...]

    return pl.pallas_call(
        body,
        out_shape=jax.ShapeDtypeStruct((A_ROWS, 128), jnp.float32),
    )(tb2, inf2)


def _tc_venue(vsum_p, vcnt_p, beta2):
    def body(vs_ref, vc_ref, b_ref, out_ref):
        s = jnp.sum(vs_ref[...], axis=0, keepdims=True)
        cnt = jnp.sum(vc_ref[...], axis=0, keepdims=True)
        out_ref[...] = b_ref[...] * s / jnp.maximum(cnt, 1.0)

    return pl.pallas_call(
        body,
        out_shape=jax.ShapeDtypeStruct((1, V_PAD), jnp.float32),
    )(vsum_p, vcnt_p, beta2)


def _tc_final(expo3, s2, inf2, it2, u2, tn2, dt2):
    def body(e_ref, s_ref, i_ref, t_ref, u_ref, tn_ref, dt_ref,
             sus_o, inf_o, it_o, sym_o, nip_o):
        expo = jnp.sum(e_ref[...], axis=0)
        s = s_ref[...]
        infected = i_ref[...]
        itime = t_ref[...]
        u = u_ref[...]
        tn = tn_ref[0, 0]
        dt = dt_ref[0, 0]

        nip = jnp.exp(-dt * s * expo)
        p = jnp.clip(nip, 1e-6, 1.0 - 1e-6)
        a = (jnp.log(p) - jnp.log(-jnp.log(u))) / TAU_C
        b = (jnp.log1p(-p) - jnp.log(-jnp.log(1.0 - u))) / TAU_C
        m = jnp.maximum(a, b)
        ea = jnp.exp(a - m)
        eb = jnp.exp(b - m)
        new_inf = eb / (ea + eb)

        sus_o[...] = jnp.maximum(0.0, s - new_inf)
        inf_o[...] = infected + new_inf
        itn = jnp.where(new_inf > 0.5, tn, itime)
        it_o[...] = itn
        sym_o[...] = (infected + new_inf) * jnp.exp(-(tn - itn))
        nip_o[...] = nip

    shp = jax.ShapeDtypeStruct((A_ROWS, 128), jnp.float32)
    return pl.pallas_call(
        body,
        out_shape=[shp, shp, shp, shp, shp],
    )(expo3, s2, inf2, it2, u2, tn2, dt2)


def kernel(susceptibility, is_infected, infection_time, transmission_base,
           edge_agent, edge_venue, beta, noise_u, timer_now, delta_time):
    f32 = jnp.float32

    def pad_a(x, v):
        return jnp.concatenate(
            [x.astype(f32), jnp.full((A_PAD - N_A,), v, f32)]
        ).reshape(A_ROWS, 128)

    tb2 = pad_a(transmission_base, 0.0)
    inf2 = pad_a(is_infected, 0.0)
    beta2 = jnp.concatenate(
        [beta, jnp.zeros((V_PAD - N_V,), f32)]).reshape(1, V_PAD)

    trans_p = _tc_trans(tb2, inf2).reshape(A_PAD)
    vsum_p, vcnt_p = _sc_pass1(trans_p, edge_agent, edge_venue)
    vval = _tc_venue(vsum_p, vcnt_p, beta2)
    expo_parts = _sc_pass2(vval.reshape(V_PAD), edge_agent, edge_venue)

    s2 = pad_a(susceptibility, 0.0)
    it2 = pad_a(infection_time, 0.0)
    u2 = pad_a(noise_u, 0.5)
    expo3 = expo_parts
    tn2 = timer_now.astype(f32).reshape(1, 1)
    dt2 = delta_time.astype(f32).reshape(1, 1)

    sus, isi, itn, sym, nip = _tc_final(expo3, s2, inf2, it2, u2, tn2, dt2)
    flat = lambda x: x.reshape(A_PAD)[:N_A]
    return flat(sus), flat(isi), flat(itn), flat(sym), flat(nip)
